# baseline (speedup 1.0000x reference)
import numpy as np
import jax
import jax.numpy as jnp
from jax.experimental import pallas as pl
from jax.experimental.pallas import tpu as pltpu

_HIDDEN = 128
_FILT = 128
_NG = 50
_NI = 6
_CUT = 10.0
_MAXZ = 100
_N = 64
_LN2 = 0.6931471805599453

_OFFS = np.linspace(0.0, _CUT, _NG).astype(np.float32)
_COEFF = float(-0.5 / (_OFFS[1] - _OFFS[0]) ** 2)

_HI = jax.lax.Precision.HIGHEST


def _ssp(x):
    return jnp.maximum(x, 0.0) + jnp.log1p(jnp.exp(-jnp.abs(x))) - _LN2


def _pair_kernel(zc_ref, pos_ref, emb_ref, w1_ref, b1_ref, w2_ref, b2_ref,
                 l1_ref, l2_ref, bl2_ref, l_ref, bl_ref,
                 o1_ref, bo1_ref, o2_ref, bo2_ref,
                 h1w_ref, h1b_ref, h2w_ref, h2b_ref, out_ref):
    f32 = jnp.float32
    N = _N
    NN = N * N
    offs = (jax.lax.broadcasted_iota(jnp.int32, (1, _NG), 1).astype(f32)
            * np.float32(_CUT / (_NG - 1)))
    pos = pos_ref[0]

    pid = jax.lax.broadcasted_iota(jnp.int32, (NN, 1), 0)
    same = (pid // N) == (pid % N)

    u_list = []
    c_list = []
    for c in range(2):
        p = pos[c]
        pi = jnp.broadcast_to(p.reshape(N, 1, 3), (N, N, 3)).reshape(NN, 3)
        pj = jnp.broadcast_to(p.reshape(1, N, 3), (N, N, 3)).reshape(NN, 3)
        diff = pi - pj
        d = jnp.sqrt(jnp.sum(diff * diff, axis=1, keepdims=True) + 1e-12)
        maskf = jnp.where((d < _CUT) & (~same), 1.0, 0.0).astype(f32)
        cc = 0.5 * (jnp.cos(d * (np.pi / _CUT)) + 1.0) * maskf
        u_list.append(_COEFF * (d - offs) ** 2)
        c_list.append(cc)
    rbf = jnp.exp(jnp.concatenate(u_list, axis=1))
    ccat = jnp.concatenate(
        [jnp.broadcast_to(c_list[0], (NN, _FILT)),
         jnp.broadcast_to(c_list[1], (NN, _FILT))], axis=1)

    zc = zc_ref[0]
    ioz = jax.lax.broadcasted_iota(jnp.int32, (N, _MAXZ), 1)
    ohc = jnp.concatenate(
        [(zc[0] == ioz).astype(f32), (zc[1] == ioz).astype(f32)], axis=1)
    h = jax.lax.dot_general(ohc, emb_ref[:, :], (((1,), (0,)), ((), ())),
                            preferred_element_type=f32, precision=_HI)

    for i in range(_NI):
        xj = jnp.dot(h, l1_ref[i], preferred_element_type=f32)
        w = _ssp(jnp.dot(rbf, w1_ref[i], preferred_element_type=f32) + b1_ref[i])
        w = jnp.dot(w, w2_ref[i], preferred_element_type=f32) + b2_ref[i]
        w = w * ccat
        agg = jnp.sum(w.reshape(N, N, 2 * _FILT) * xj[None, :, :], axis=1)
        m = _ssp(jnp.dot(agg, l2_ref[i], preferred_element_type=f32) + bl2_ref[i])
        m = jnp.dot(m, l_ref[i], preferred_element_type=f32) + bl_ref[i]
        h = h + m

    o = _ssp(jnp.dot(h, o1_ref[:, :], preferred_element_type=f32) + bo1_ref[:, :])
    s = jnp.sum(o, axis=0, keepdims=True)
    e = (jnp.dot(s, o2_ref[:, :], preferred_element_type=f32, precision=_HI)
         + float(N) * bo2_ref[:, :])
    y = jnp.maximum(
        jnp.dot(e, h1w_ref[:, :], preferred_element_type=f32, precision=_HI)
        + h1b_ref[:, :], 0.0)
    y = (jnp.dot(y, h2w_ref[:, :], preferred_element_type=f32, precision=_HI)
         + h2b_ref[:, :])
    out_ref[:, :, :] = y.reshape(1, 1, 1)


def _bdiag(a, b):
    ka, na = a.shape
    kb, nb = b.shape
    return jnp.concatenate(
        [jnp.concatenate([a, jnp.zeros((ka, nb), jnp.float32)], 1),
         jnp.concatenate([jnp.zeros((kb, na), jnp.float32), b], 1)], 0)


def kernel(z, pos, params1, params2, head):
    B = z.shape[0]
    zq = z.reshape(B, 2, _N, 1).astype(jnp.int32)
    pq = pos.reshape(B, 2, _N, 3).astype(jnp.float32)

    i1 = params1["inter"]
    i2 = params2["inter"]
    W1s = jnp.stack([_bdiag(i1[i]["mlp1"]["w"], i2[i]["mlp1"]["w"]) for i in range(_NI)])
    B1s = jnp.stack([jnp.concatenate([i1[i]["mlp1"]["b"], i2[i]["mlp1"]["b"]])[None, :] for i in range(_NI)])
    W2s = jnp.stack([_bdiag(i1[i]["mlp2"]["w"], i2[i]["mlp2"]["w"]) for i in range(_NI)])
    B2s = jnp.stack([jnp.concatenate([i1[i]["mlp2"]["b"], i2[i]["mlp2"]["b"]])[None, :] for i in range(_NI)])
    L1s = jnp.stack([_bdiag(i1[i]["lin1"]["w"], i2[i]["lin1"]["w"]) for i in range(_NI)])
    L2s = jnp.stack([_bdiag(i1[i]["lin2"]["w"], i2[i]["lin2"]["w"]) for i in range(_NI)])
    BL2s = jnp.stack([jnp.concatenate([i1[i]["lin2"]["b"], i2[i]["lin2"]["b"]])[None, :] for i in range(_NI)])
    Ls = jnp.stack([_bdiag(i1[i]["lin"]["w"], i2[i]["lin"]["w"]) for i in range(_NI)])
    BLs = jnp.stack([jnp.concatenate([i1[i]["lin"]["b"], i2[i]["lin"]["b"]])[None, :] for i in range(_NI)])
    EMB = _bdiag(params1["embed"], params2["embed"])
    O1 = _bdiag(params1["out1"]["w"], params2["out1"]["w"])
    BO1 = jnp.concatenate([params1["out1"]["b"], params2["out1"]["b"]])[None, :]
    O2 = _bdiag(params1["out2"]["w"], params2["out2"]["w"])
    BO2 = jnp.concatenate([params1["out2"]["b"], params2["out2"]["b"]])[None, :]
    H1W = head["l1"]["w"]
    H1B = head["l1"]["b"][None, :]
    H2W = head["l2"]["w"]
    H2B = head["l2"]["b"][None, :]

    def full(a):
        return pl.BlockSpec(a.shape, lambda b, nd=a.ndim: (0,) * nd)

    consts = (EMB, W1s, B1s, W2s, B2s, L1s, L2s, BL2s, Ls, BLs,
              O1, BO1, O2, BO2, H1W, H1B, H2W, H2B)
    out = pl.pallas_call(
        _pair_kernel,
        grid=(B,),
        in_specs=[
            pl.BlockSpec((1, 2, _N, 1), lambda b: (b, 0, 0, 0)),
            pl.BlockSpec((1, 2, _N, 3), lambda b: (b, 0, 0, 0)),
        ] + [full(a) for a in consts],
        out_specs=pl.BlockSpec((1, 1, 1), lambda b: (b, 0, 0)),
        out_shape=jax.ShapeDtypeStruct((B, 1, 1), jnp.float32),
        compiler_params=pltpu.CompilerParams(dimension_semantics=("arbitrary",)),
    )(zq, pq, *consts)
    return out.reshape(B, 1)
